# Initial kernel scaffold; baseline (speedup 1.0000x reference)
import jax
import jax.numpy as jnp
from jax.experimental import pallas as pl


def kernel(x, edge_index, edge_attr, W1, b1, W2, b2, W3, b3, W4, b4, W5, b5):
    raise NotImplementedError("write your pallas kernel here")


# speedup vs baseline: 6.0841x; 6.0841x over previous
import functools

import jax
import jax.numpy as jnp
from jax import lax
from jax.experimental import pallas as pl
from jax.experimental.pallas import tpu as pltpu
from jax.experimental.pallas import tpu_sc as plsc

N = 10000
E = 320000
D_FEAT = 128
D_AUG = 32

NC = 2
NS = 16
NW = NC * NS
EPW = E // NW
CHUNK = 80
NCHUNK = EPW // CHUNK
ZROWS = 78
FLUSH = 624
TAIL0 = NS * FLUSH
TAILR = N - TAIL0


def _sc_body(x_hbm, dst_hbm, src_hbm, ea_hbm, accx_out, acca_out,
             idx_dst, idx_src, rows, abuf, zbuf, zbuf_a, acc_x, acc_a, gsem):
    cid = lax.axis_index("c")
    sid = lax.axis_index("s")
    wid = cid * NS + sid

    def zero_row(i, _):
        for j in range(D_FEAT // 16):
            zbuf[i, pl.ds(j * 16, 16)] = jnp.zeros((16,), jnp.float32)
        for j in range(D_AUG // 16):
            zbuf_a[i, pl.ds(j * 16, 16)] = jnp.zeros((16,), jnp.float32)
        return 0

    lax.fori_loop(0, ZROWS, zero_row, 0)
    row0 = sid * FLUSH
    for r in range(FLUSH // ZROWS):
        pltpu.sync_copy(zbuf, acc_x.at[pl.ds(row0 + r * ZROWS, ZROWS)])
        pltpu.sync_copy(zbuf_a, acc_a.at[pl.ds(row0 + r * ZROWS, ZROWS)])

    @pl.when(sid == NS - 1)
    def _zero_tail():
        pltpu.sync_copy(zbuf.at[pl.ds(0, TAILR)], acc_x.at[pl.ds(TAIL0, TAILR)])
        pltpu.sync_copy(zbuf_a.at[pl.ds(0, TAILR)],
                        acc_a.at[pl.ds(TAIL0, TAILR)])

    onecol = jnp.where(lax.iota(jnp.int32, 16) == 0,
                       jnp.float32(1.0), jnp.float32(0.0))

    def init_abuf(i, _):
        abuf[i, pl.ds(16, 16)] = onecol
        return 0

    lax.fori_loop(0, CHUNK, init_abuf, 0)

    plsc.subcore_barrier()

    def chunk_body(k, _):
        base = wid * EPW + k * CHUNK
        pltpu.sync_copy(dst_hbm.at[pl.ds(base, CHUNK)], idx_dst)
        pltpu.sync_copy(src_hbm.at[pl.ds(base, CHUNK)], idx_src)
        pltpu.async_copy(x_hbm.at[idx_src], rows, gsem).wait()
        pltpu.sync_copy(ea_hbm.at[pl.ds(base, CHUNK)],
                        abuf.at[:, pl.ds(0, 16)])
        pltpu.sync_copy(rows, acc_x.at[idx_dst], add=True)
        pltpu.sync_copy(abuf, acc_a.at[idx_dst], add=True)
        return 0

    lax.fori_loop(0, NCHUNK, chunk_body, 0)

    plsc.subcore_barrier()

    pltpu.sync_copy(acc_x.at[pl.ds(row0, FLUSH)],
                    accx_out.at[cid, pl.ds(row0, FLUSH)])
    pltpu.sync_copy(acc_a.at[pl.ds(row0, FLUSH)],
                    acca_out.at[cid, pl.ds(row0, FLUSH)])

    @pl.when(sid == NS - 1)
    def _flush_tail():
        pltpu.sync_copy(acc_x.at[pl.ds(TAIL0, TAILR)],
                        accx_out.at[cid, pl.ds(TAIL0, TAILR)])
        pltpu.sync_copy(acc_a.at[pl.ds(TAIL0, TAILR)],
                        acca_out.at[cid, pl.ds(TAIL0, TAILR)])


_sc_segsum = pl.kernel(
    _sc_body,
    out_type=(
        jax.ShapeDtypeStruct((NC, N, D_FEAT), jnp.float32),
        jax.ShapeDtypeStruct((NC, N, D_AUG), jnp.float32),
    ),
    mesh=plsc.VectorSubcoreMesh(core_axis_name="c", subcore_axis_name="s"),
    scratch_types=[
        pltpu.VMEM((CHUNK,), jnp.int32),
        pltpu.VMEM((CHUNK,), jnp.int32),
        pltpu.VMEM((CHUNK, D_FEAT), jnp.float32),
        pltpu.VMEM((CHUNK, D_AUG), jnp.float32),
        pltpu.VMEM((ZROWS, D_FEAT), jnp.float32),
        pltpu.VMEM((ZROWS, D_AUG), jnp.float32),
        pltpu.VMEM_SHARED((N, D_FEAT), jnp.float32),
        pltpu.VMEM_SHARED((N, D_AUG), jnp.float32),
        pltpu.SemaphoreType.DMA,
    ],
    compiler_params=pltpu.CompilerParams(use_tc_tiling_on_sc=False),
)


def _tc_body(x_ref, ax_ref, aa_ref, w1_ref, w2_ref, w5_ref, b1_ref, o_ref):
    acc = jnp.dot(x_ref[...], w1_ref[...], preferred_element_type=jnp.float32)
    acc += jnp.dot(ax_ref[0] + ax_ref[1], w2_ref[...],
                   preferred_element_type=jnp.float32)
    acc += jnp.dot(aa_ref[0] + aa_ref[1], w5_ref[...],
                   preferred_element_type=jnp.float32)
    o_ref[...] = acc + b1_ref[...]


ROW_BLK = 1000


def _tc_combine(x, accx, acca, W1, W2, W5aug, b1):
    return pl.pallas_call(
        _tc_body,
        out_shape=jax.ShapeDtypeStruct((N, D_FEAT), jnp.float32),
        grid=(N // ROW_BLK,),
        in_specs=[
            pl.BlockSpec((ROW_BLK, D_FEAT), lambda i: (i, 0)),
            pl.BlockSpec((NC, ROW_BLK, D_FEAT), lambda i: (0, i, 0)),
            pl.BlockSpec((NC, ROW_BLK, D_AUG), lambda i: (0, i, 0)),
            pl.BlockSpec((D_FEAT, D_FEAT), lambda i: (0, 0)),
            pl.BlockSpec((D_FEAT, D_FEAT), lambda i: (0, 0)),
            pl.BlockSpec((D_AUG, D_FEAT), lambda i: (0, 0)),
            pl.BlockSpec((1, D_FEAT), lambda i: (0, 0)),
        ],
        out_specs=pl.BlockSpec((ROW_BLK, D_FEAT), lambda i: (i, 0)),
    )(x, accx, acca, W1, W2, W5aug, b1)


def kernel(x, edge_index, edge_attr, W1, b1, W2, b2, W3, b3, W4, b4, W5, b5):
    dst = edge_index[0]
    src = edge_index[1]
    accx, acca = _sc_segsum(x, dst, src, edge_attr)
    W5aug = jnp.zeros((D_AUG, D_FEAT), jnp.float32).at[0:16].set(W5).at[16].set(b2 + b5)
    return _tc_combine(x, accx, acca, W1, W2, W5aug, b1.reshape(1, D_FEAT))


# speedup vs baseline: 6.9119x; 1.1360x over previous
import jax
import jax.numpy as jnp
from jax import lax
from jax.experimental import pallas as pl
from jax.experimental.pallas import tpu as pltpu
from jax.experimental.pallas import tpu_sc as plsc

N = 10000
E = 320000
D_FEAT = 128
D_AUG = 32

NC = 2
NS = 16
NW = NC * NS
EPW = E // NW
CHUNK = 50
NCHUNK = EPW // CHUNK
SLAB = 10
NSLAB = NCHUNK // SLAB
NBODY = NCHUNK // (2 * SLAB)
ZCOPY = 48
FLUSH = 624
TAIL0 = NS * FLUSH
TAILR = N - TAIL0


def _sc_body(x_hbm, didx_hbm, ea_hbm, accx_out, acca_out,
             slab0, slab1, rows0, rows1, abuf0, abuf1,
             acc_x, acc_a, isem0, isem1, osem0, osem1, ssem0, ssem1):
    cid = lax.axis_index("c")
    sid = lax.axis_index("s")
    wid = cid * NS + sid

    slab = (slab0, slab1)
    rows = (rows0, rows1)
    abuf = (abuf0, abuf1)
    isem = (isem0, isem1)
    osem = (osem0, osem1)
    ssem = (ssem0, ssem1)

    def zero_row(i, _):
        for j in range(D_FEAT // 16):
            rows0[i, pl.ds(j * 16, 16)] = jnp.zeros((16,), jnp.float32)
        for j in range(D_AUG // 16):
            abuf0[i, pl.ds(j * 16, 16)] = jnp.zeros((16,), jnp.float32)
        return 0

    lax.fori_loop(0, CHUNK, zero_row, 0)
    row0 = sid * FLUSH
    for r in range(FLUSH // ZCOPY):
        pltpu.sync_copy(rows0.at[pl.ds(0, ZCOPY)],
                        acc_x.at[pl.ds(row0 + r * ZCOPY, ZCOPY)])
        pltpu.sync_copy(abuf0.at[pl.ds(0, ZCOPY)],
                        acc_a.at[pl.ds(row0 + r * ZCOPY, ZCOPY)])

    @pl.when(sid == NS - 1)
    def _zero_tail():
        pltpu.sync_copy(rows0.at[pl.ds(0, TAILR)],
                        acc_x.at[pl.ds(TAIL0, TAILR)])
        pltpu.sync_copy(abuf0.at[pl.ds(0, TAILR)],
                        acc_a.at[pl.ds(TAIL0, TAILR)])

    onecol = jnp.where(lax.iota(jnp.int32, 16) == 0,
                       jnp.float32(1.0), jnp.float32(0.0))

    def init_abuf(i, _):
        abuf0[i, pl.ds(16, 16)] = onecol
        abuf1[i, pl.ds(16, 16)] = onecol
        return 0

    lax.fori_loop(0, CHUNK, init_abuf, 0)

    plsc.subcore_barrier()

    def issue_in(c, b, S, j):
        pltpu.async_copy(x_hbm.at[S.at[2 * j + 1]], rows[b], isem[b])
        pltpu.async_copy(ea_hbm.at[wid * NCHUNK + c],
                         abuf[b].at[:, pl.ds(0, 16)], isem[b])

    def wait_in(c, b, S, j):
        pltpu.make_async_copy(x_hbm.at[S.at[2 * j + 1]], rows[b],
                              isem[b]).wait()
        pltpu.make_async_copy(ea_hbm.at[wid * NCHUNK + c],
                              abuf[b].at[:, pl.ds(0, 16)], isem[b]).wait()

    def issue_out(b, S, j):
        pltpu.async_copy(rows[b], acc_x.at[S.at[2 * j]], osem[b], add=True)
        pltpu.async_copy(abuf[b], acc_a.at[S.at[2 * j]], osem[b], add=True)

    def wait_out(b):
        pltpu.make_async_copy(rows[b], acc_x.at[slab0.at[0]], osem[b]).wait()
        pltpu.make_async_copy(abuf[b], acc_a.at[slab0.at[0]], osem[b]).wait()

    def issue_slab(s, sb):
        pltpu.async_copy(didx_hbm.at[wid * NSLAB + s], slab[sb], ssem[sb])

    def wait_slab(s, sb):
        pltpu.make_async_copy(didx_hbm.at[wid * NSLAB + s], slab[sb],
                              ssem[sb]).wait()

    pltpu.sync_copy(didx_hbm.at[wid * NSLAB], slab0)
    issue_in(0, 0, slab0, 0)

    def body(i, _):
        for js in range(2):
            S = slab[js]
            for j in range(SLAB):
                c = 20 * i + 10 * js + j
                b = j % 2
                wait_in(c, b, S, j)
                issue_out(b, S, j)
                if js == 0 and j == 0:
                    @pl.when(i > 0)
                    def _():
                        wait_out(1 - b)
                else:
                    wait_out(1 - b)
                if js == 0 and j == 1:
                    issue_slab(2 * i + 1, 1)
                if js == 1 and j == 1:
                    @pl.when(i < NBODY - 1)
                    def _():
                        issue_slab(2 * i + 2, 0)
                if j == SLAB - 1:
                    if js == 0:
                        wait_slab(2 * i + 1, 1)
                        issue_in(c + 1, 1 - b, slab1, 0)
                    else:
                        @pl.when(i < NBODY - 1)
                        def _():
                            wait_slab(2 * i + 2, 0)
                            issue_in(c + 1, 1 - b, slab0, 0)
                else:
                    issue_in(c + 1, 1 - b, S, j + 1)
        return 0

    lax.fori_loop(0, NBODY, body, 0)

    wait_out(1)

    plsc.subcore_barrier()

    pltpu.sync_copy(acc_x.at[pl.ds(row0, FLUSH)],
                    accx_out.at[cid, pl.ds(row0, FLUSH)])
    pltpu.sync_copy(acc_a.at[pl.ds(row0, FLUSH)],
                    acca_out.at[cid, pl.ds(row0, FLUSH)])

    @pl.when(sid == NS - 1)
    def _flush_tail():
        pltpu.sync_copy(acc_x.at[pl.ds(TAIL0, TAILR)],
                        accx_out.at[cid, pl.ds(TAIL0, TAILR)])
        pltpu.sync_copy(acc_a.at[pl.ds(TAIL0, TAILR)],
                        acca_out.at[cid, pl.ds(TAIL0, TAILR)])


_sc_segsum = pl.kernel(
    _sc_body,
    out_type=(
        jax.ShapeDtypeStruct((NC, N, D_FEAT), jnp.float32),
        jax.ShapeDtypeStruct((NC, N, D_AUG), jnp.float32),
    ),
    mesh=plsc.VectorSubcoreMesh(core_axis_name="c", subcore_axis_name="s"),
    scratch_types=[
        pltpu.VMEM((2 * SLAB, CHUNK), jnp.int32),
        pltpu.VMEM((2 * SLAB, CHUNK), jnp.int32),
        pltpu.VMEM((CHUNK, D_FEAT), jnp.float32),
        pltpu.VMEM((CHUNK, D_FEAT), jnp.float32),
        pltpu.VMEM((CHUNK, D_AUG), jnp.float32),
        pltpu.VMEM((CHUNK, D_AUG), jnp.float32),
        pltpu.VMEM_SHARED((N, D_FEAT), jnp.float32),
        pltpu.VMEM_SHARED((N, D_AUG), jnp.float32),
        pltpu.SemaphoreType.DMA,
        pltpu.SemaphoreType.DMA,
        pltpu.SemaphoreType.DMA,
        pltpu.SemaphoreType.DMA,
        pltpu.SemaphoreType.DMA,
        pltpu.SemaphoreType.DMA,
    ],
    compiler_params=pltpu.CompilerParams(use_tc_tiling_on_sc=False),
)


def _tc_body(x_ref, ax_ref, aa_ref, w1_ref, w2_ref, w5_ref, b1_ref, o_ref):
    acc = jnp.dot(x_ref[...], w1_ref[...], preferred_element_type=jnp.float32)
    acc += jnp.dot(ax_ref[0] + ax_ref[1], w2_ref[...],
                   preferred_element_type=jnp.float32)
    acc += jnp.dot(aa_ref[0] + aa_ref[1], w5_ref[...],
                   preferred_element_type=jnp.float32)
    o_ref[...] = acc + b1_ref[...]


ROW_BLK = 1000


def _tc_combine(x, accx, acca, W1, W2, W5aug, b1):
    return pl.pallas_call(
        _tc_body,
        out_shape=jax.ShapeDtypeStruct((N, D_FEAT), jnp.float32),
        grid=(N // ROW_BLK,),
        in_specs=[
            pl.BlockSpec((ROW_BLK, D_FEAT), lambda i: (i, 0)),
            pl.BlockSpec((NC, ROW_BLK, D_FEAT), lambda i: (0, i, 0)),
            pl.BlockSpec((NC, ROW_BLK, D_AUG), lambda i: (0, i, 0)),
            pl.BlockSpec((D_FEAT, D_FEAT), lambda i: (0, 0)),
            pl.BlockSpec((D_FEAT, D_FEAT), lambda i: (0, 0)),
            pl.BlockSpec((D_AUG, D_FEAT), lambda i: (0, 0)),
            pl.BlockSpec((1, D_FEAT), lambda i: (0, 0)),
        ],
        out_specs=pl.BlockSpec((ROW_BLK, D_FEAT), lambda i: (i, 0)),
    )(x, accx, acca, W1, W2, W5aug, b1)


def kernel(x, edge_index, edge_attr, W1, b1, W2, b2, W3, b3, W4, b4, W5, b5):
    dst = edge_index[0].reshape(NW, NSLAB, SLAB, 1, CHUNK)
    src = edge_index[1].reshape(NW, NSLAB, SLAB, 1, CHUNK)
    didx = jnp.concatenate([dst, src], axis=3).reshape(
        NW * NSLAB, 2 * SLAB, CHUNK)
    ea3 = edge_attr.reshape(NW * NCHUNK, CHUNK, 16)
    accx, acca = _sc_segsum(x, didx, ea3)
    W5aug = jnp.zeros((D_AUG, D_FEAT), jnp.float32).at[0:16].set(W5).at[16].set(b2 + b5)
    return _tc_combine(x, accx, acca, W1, W2, W5aug, b1.reshape(1, D_FEAT))


# speedup vs baseline: 7.8900x; 1.1415x over previous
import jax
import jax.numpy as jnp
from jax import lax
from jax.experimental import pallas as pl
from jax.experimental.pallas import tpu as pltpu
from jax.experimental.pallas import tpu_sc as plsc

N = 10000
E = 320000
D_FEAT = 128
D_EDGE = 16
D_AUG = 32

NC = 2
NS = 16
NW = NC * NS
EPW = E // NW
CHUNK = 40
EROWS = CHUNK // 8
NCHUNK = EPW // CHUNK
SLAB = 5
NSLAB = NCHUNK // SLAB
NBODY = NCHUNK // (2 * SLAB)
FLUSH = 624
TAIL0 = NS * FLUSH
TAILR = N - TAIL0


def _sc_body(x_hbm, didx_hbm, ea_hbm, accx_out, acca_out,
             slab0, slab1, rows0, rows1, ebuf0, ebuf1, abuf0, abuf1,
             acc_x, acc_a, isem0, isem1, osem0, osem1, ssem0, ssem1):
    cid = lax.axis_index("c")
    sid = lax.axis_index("s")
    wid = cid * NS + sid

    slab = (slab0, slab1)
    rows = (rows0, rows1)
    ebuf = (ebuf0, ebuf1)
    abuf = (abuf0, abuf1)
    isem = (isem0, isem1)
    osem = (osem0, osem1)
    ssem = (ssem0, ssem1)

    def zero_row(i, _):
        for j in range(D_FEAT // 16):
            rows0[i, pl.ds(j * 16, 16)] = jnp.zeros((16,), jnp.float32)
        for j in range(D_AUG // 16):
            abuf0[i, pl.ds(j * 16, 16)] = jnp.zeros((16,), jnp.float32)
        return 0

    lax.fori_loop(0, CHUNK, zero_row, 0)
    row0 = sid * FLUSH
    for r in range(FLUSH // CHUNK):
        pltpu.sync_copy(rows0, acc_x.at[pl.ds(row0 + r * CHUNK, CHUNK)])
        pltpu.sync_copy(abuf0, acc_a.at[pl.ds(row0 + r * CHUNK, CHUNK)])
    zrem = FLUSH - (FLUSH // CHUNK) * CHUNK
    pltpu.sync_copy(rows0.at[pl.ds(0, zrem)],
                    acc_x.at[pl.ds(row0 + FLUSH - zrem, zrem)])
    pltpu.sync_copy(abuf0.at[pl.ds(0, zrem)],
                    acc_a.at[pl.ds(row0 + FLUSH - zrem, zrem)])

    @pl.when(sid == NS - 1)
    def _zero_tail():
        pltpu.sync_copy(rows0.at[pl.ds(0, TAILR)],
                        acc_x.at[pl.ds(TAIL0, TAILR)])
        pltpu.sync_copy(abuf0.at[pl.ds(0, TAILR)],
                        acc_a.at[pl.ds(TAIL0, TAILR)])

    onecol = jnp.where(lax.iota(jnp.int32, 16) == 0,
                       jnp.float32(1.0), jnp.float32(0.0))

    def init_abuf(i, _):
        abuf0[i, pl.ds(16, 16)] = onecol
        abuf1[i, pl.ds(16, 16)] = onecol
        return 0

    lax.fori_loop(0, CHUNK, init_abuf, 0)

    plsc.subcore_barrier()

    def issue_in(c, b, S, j):
        pltpu.async_copy(x_hbm.at[S.at[2 * j + 1]], rows[b], isem[b])
        pltpu.async_copy(ea_hbm.at[pl.ds((wid * NCHUNK + c) * EROWS, EROWS)],
                         ebuf[b], isem[b])

    def wait_in(c, b, S, j):
        pltpu.make_async_copy(x_hbm.at[S.at[2 * j + 1]], rows[b],
                              isem[b]).wait()
        pltpu.make_async_copy(ea_hbm.at[pl.ds((wid * NCHUNK + c) * EROWS,
                                              EROWS)],
                              ebuf[b], isem[b]).wait()

    def unpack_attr(b):
        eb = ebuf[b]
        ab = abuf[b]
        for i in range(CHUNK):
            ab[i, pl.ds(0, D_EDGE)] = eb[i // 8, pl.ds((i % 8) * D_EDGE,
                                                       D_EDGE)]

    def issue_out(b, S, j):
        pltpu.async_copy(rows[b], acc_x.at[S.at[2 * j]], osem[b], add=True)
        pltpu.async_copy(abuf[b], acc_a.at[S.at[2 * j]], osem[b], add=True)

    def wait_out(b):
        pltpu.make_async_copy(rows[b], acc_x.at[slab0.at[0]], osem[b]).wait()
        pltpu.make_async_copy(abuf[b], acc_a.at[slab0.at[0]], osem[b]).wait()

    def issue_slab(s, sb):
        pltpu.async_copy(didx_hbm.at[wid * NSLAB + s], slab[sb], ssem[sb])

    def wait_slab(s, sb):
        pltpu.make_async_copy(didx_hbm.at[wid * NSLAB + s], slab[sb],
                              ssem[sb]).wait()

    pltpu.sync_copy(didx_hbm.at[wid * NSLAB], slab0)
    issue_in(0, 0, slab0, 0)

    def body(i, _):
        for js in range(2):
            S = slab[js]
            for j in range(SLAB):
                c = 10 * i + 5 * js + j
                b = (js + j) % 2
                wait_in(c, b, S, j)
                unpack_attr(b)
                issue_out(b, S, j)
                if js == 0 and j == 0:
                    @pl.when(i > 0)
                    def _():
                        wait_out(1 - b)
                else:
                    wait_out(1 - b)
                if js == 0 and j == 1:
                    issue_slab(2 * i + 1, 1)
                if js == 1 and j == 1:
                    @pl.when(i < NBODY - 1)
                    def _():
                        issue_slab(2 * i + 2, 0)
                if j == SLAB - 1:
                    if js == 0:
                        wait_slab(2 * i + 1, 1)
                        issue_in(c + 1, 1 - b, slab1, 0)
                    else:
                        @pl.when(i < NBODY - 1)
                        def _():
                            wait_slab(2 * i + 2, 0)
                            issue_in(c + 1, 1 - b, slab0, 0)
                else:
                    issue_in(c + 1, 1 - b, S, j + 1)
        return 0

    lax.fori_loop(0, NBODY, body, 0)

    wait_out(1)

    plsc.subcore_barrier()

    pltpu.sync_copy(acc_x.at[pl.ds(row0, FLUSH)],
                    accx_out.at[cid, pl.ds(row0, FLUSH)])
    pltpu.sync_copy(acc_a.at[pl.ds(row0, FLUSH)],
                    acca_out.at[cid, pl.ds(row0, FLUSH)])

    @pl.when(sid == NS - 1)
    def _flush_tail():
        pltpu.sync_copy(acc_x.at[pl.ds(TAIL0, TAILR)],
                        accx_out.at[cid, pl.ds(TAIL0, TAILR)])
        pltpu.sync_copy(acc_a.at[pl.ds(TAIL0, TAILR)],
                        acca_out.at[cid, pl.ds(TAIL0, TAILR)])


_sc_segsum = pl.kernel(
    _sc_body,
    out_type=(
        jax.ShapeDtypeStruct((NC, N, D_FEAT), jnp.float32),
        jax.ShapeDtypeStruct((NC, N, D_AUG), jnp.float32),
    ),
    mesh=plsc.VectorSubcoreMesh(core_axis_name="c", subcore_axis_name="s"),
    scratch_types=[
        pltpu.VMEM((2 * SLAB, CHUNK), jnp.int32),
        pltpu.VMEM((2 * SLAB, CHUNK), jnp.int32),
        pltpu.VMEM((CHUNK, D_FEAT), jnp.float32),
        pltpu.VMEM((CHUNK, D_FEAT), jnp.float32),
        pltpu.VMEM((EROWS, D_FEAT), jnp.float32),
        pltpu.VMEM((EROWS, D_FEAT), jnp.float32),
        pltpu.VMEM((CHUNK, D_AUG), jnp.float32),
        pltpu.VMEM((CHUNK, D_AUG), jnp.float32),
        pltpu.VMEM_SHARED((N, D_FEAT), jnp.float32),
        pltpu.VMEM_SHARED((N, D_AUG), jnp.float32),
        pltpu.SemaphoreType.DMA,
        pltpu.SemaphoreType.DMA,
        pltpu.SemaphoreType.DMA,
        pltpu.SemaphoreType.DMA,
        pltpu.SemaphoreType.DMA,
        pltpu.SemaphoreType.DMA,
    ],
    compiler_params=pltpu.CompilerParams(use_tc_tiling_on_sc=False),
)


def _tc_body(x_ref, ax_ref, aa_ref, w1_ref, w2_ref, w5_ref, b1_ref, o_ref):
    acc = jnp.dot(x_ref[...], w1_ref[...], preferred_element_type=jnp.float32)
    acc += jnp.dot(ax_ref[0] + ax_ref[1], w2_ref[...],
                   preferred_element_type=jnp.float32)
    acc += jnp.dot(aa_ref[0] + aa_ref[1], w5_ref[...],
                   preferred_element_type=jnp.float32)
    o_ref[...] = acc + b1_ref[...]


ROW_BLK = 1000


def _tc_combine(x, accx, acca, W1, W2, W5aug, b1):
    return pl.pallas_call(
        _tc_body,
        out_shape=jax.ShapeDtypeStruct((N, D_FEAT), jnp.float32),
        grid=(N // ROW_BLK,),
        in_specs=[
            pl.BlockSpec((ROW_BLK, D_FEAT), lambda i: (i, 0)),
            pl.BlockSpec((NC, ROW_BLK, D_FEAT), lambda i: (0, i, 0)),
            pl.BlockSpec((NC, ROW_BLK, D_AUG), lambda i: (0, i, 0)),
            pl.BlockSpec((D_FEAT, D_FEAT), lambda i: (0, 0)),
            pl.BlockSpec((D_FEAT, D_FEAT), lambda i: (0, 0)),
            pl.BlockSpec((D_AUG, D_FEAT), lambda i: (0, 0)),
            pl.BlockSpec((1, D_FEAT), lambda i: (0, 0)),
        ],
        out_specs=pl.BlockSpec((ROW_BLK, D_FEAT), lambda i: (i, 0)),
    )(x, accx, acca, W1, W2, W5aug, b1)


def kernel(x, edge_index, edge_attr, W1, b1, W2, b2, W3, b3, W4, b4, W5, b5):
    dst = edge_index[0].reshape(NW, NSLAB, SLAB, 1, CHUNK)
    src = edge_index[1].reshape(NW, NSLAB, SLAB, 1, CHUNK)
    didx = jnp.concatenate([dst, src], axis=3).reshape(
        NW * NSLAB, 2 * SLAB, CHUNK)
    eaA = edge_attr.reshape(E // 8, D_FEAT)
    accx, acca = _sc_segsum(x, didx, eaA)
    W5aug = jnp.zeros((D_AUG, D_FEAT), jnp.float32).at[0:16].set(W5).at[16].set(b2 + b5)
    return _tc_combine(x, accx, acca, W1, W2, W5aug, b1.reshape(1, D_FEAT))


# speedup vs baseline: 8.9796x; 1.1381x over previous
import jax
import jax.numpy as jnp
from jax import lax
from jax.experimental import pallas as pl
from jax.experimental.pallas import tpu as pltpu
from jax.experimental.pallas import tpu_sc as plsc

N = 10000
E = 320000
D_FEAT = 128
D_EDGE = 16
D_AUG = 32

NC = 2
NS = 16
NW = NC * NS
EPW = E // NW
CHUNK = 40
NCHUNK = EPW // CHUNK
SLAB = 5
SPAN = SLAB * CHUNK
NSLAB = NCHUNK // SLAB
NBODY = NCHUNK // (2 * SLAB)
FLUSH = 624
TAIL0 = NS * FLUSH
TAILR = N - TAIL0


def _sc_body(x_hbm, ei_hbm, ea_hbm, accx_out, acca_out,
             slab0, slab1, rows0, rows1, abuf0, abuf1,
             acc_x, acc_a, isem0, isem1, osem0, osem1, ssem0, ssem1):
    cid = lax.axis_index("c")
    sid = lax.axis_index("s")
    wid = cid * NS + sid

    slab = (slab0, slab1)
    rows = (rows0, rows1)
    abuf = (abuf0, abuf1)
    isem = (isem0, isem1)
    osem = (osem0, osem1)
    ssem = (ssem0, ssem1)

    def zero_row(i, _):
        for j in range(D_FEAT // 16):
            rows0[i, pl.ds(j * 16, 16)] = jnp.zeros((16,), jnp.float32)
        for j in range(D_AUG // 16):
            abuf0[i, pl.ds(j * 16, 16)] = jnp.zeros((16,), jnp.float32)
        return 0

    lax.fori_loop(0, CHUNK, zero_row, 0)
    row0 = sid * FLUSH
    for r in range(FLUSH // CHUNK):
        pltpu.sync_copy(rows0, acc_x.at[pl.ds(row0 + r * CHUNK, CHUNK)])
        pltpu.sync_copy(abuf0, acc_a.at[pl.ds(row0 + r * CHUNK, CHUNK)])
    zrem = FLUSH - (FLUSH // CHUNK) * CHUNK
    pltpu.sync_copy(rows0.at[pl.ds(0, zrem)],
                    acc_x.at[pl.ds(row0 + FLUSH - zrem, zrem)])
    pltpu.sync_copy(abuf0.at[pl.ds(0, zrem)],
                    acc_a.at[pl.ds(row0 + FLUSH - zrem, zrem)])

    @pl.when(sid == NS - 1)
    def _zero_tail():
        pltpu.sync_copy(rows0.at[pl.ds(0, TAILR)],
                        acc_x.at[pl.ds(TAIL0, TAILR)])
        pltpu.sync_copy(abuf0.at[pl.ds(0, TAILR)],
                        acc_a.at[pl.ds(TAIL0, TAILR)])

    onecol = jnp.where(lax.iota(jnp.int32, 16) == 0,
                       jnp.float32(1.0), jnp.float32(0.0))

    def init_abuf(i, _):
        abuf0[i, pl.ds(16, 16)] = onecol
        abuf1[i, pl.ds(16, 16)] = onecol
        return 0

    lax.fori_loop(0, CHUNK, init_abuf, 0)

    plsc.subcore_barrier()

    def issue_in(c, b, S, j):
        pltpu.async_copy(x_hbm.at[S.at[1, pl.ds(j * CHUNK, CHUNK)]],
                         rows[b], isem[b])
        pltpu.async_copy(ea_hbm.at[pl.ds(wid * EPW + c * CHUNK, CHUNK)],
                         abuf[b].at[:, pl.ds(0, D_EDGE)], isem[b])

    def wait_in(c, b, S, j):
        pltpu.make_async_copy(x_hbm.at[S.at[1, pl.ds(j * CHUNK, CHUNK)]],
                              rows[b], isem[b]).wait()
        pltpu.make_async_copy(ea_hbm.at[pl.ds(wid * EPW + c * CHUNK, CHUNK)],
                              abuf[b].at[:, pl.ds(0, D_EDGE)], isem[b]).wait()

    def issue_out(b, S, j):
        pltpu.async_copy(rows[b], acc_x.at[S.at[0, pl.ds(j * CHUNK, CHUNK)]],
                         osem[b], add=True)
        pltpu.async_copy(abuf[b], acc_a.at[S.at[0, pl.ds(j * CHUNK, CHUNK)]],
                         osem[b], add=True)

    def wait_out(b):
        pltpu.make_async_copy(rows[b], acc_x.at[slab0.at[0, pl.ds(0, CHUNK)]],
                              osem[b]).wait()
        pltpu.make_async_copy(abuf[b], acc_a.at[slab0.at[0, pl.ds(0, CHUNK)]],
                              osem[b]).wait()

    def issue_slab(s, sb):
        base = wid * EPW + s * SPAN
        pltpu.async_copy(ei_hbm.at[0, pl.ds(base, SPAN)],
                         slab[sb].at[0], ssem[sb])
        pltpu.async_copy(ei_hbm.at[1, pl.ds(base, SPAN)],
                         slab[sb].at[1], ssem[sb])

    def wait_slab(s, sb):
        base = wid * EPW + s * SPAN
        pltpu.make_async_copy(ei_hbm.at[0, pl.ds(base, SPAN)],
                              slab[sb].at[0], ssem[sb]).wait()
        pltpu.make_async_copy(ei_hbm.at[1, pl.ds(base, SPAN)],
                              slab[sb].at[1], ssem[sb]).wait()

    issue_slab(0, 0)
    wait_slab(0, 0)
    issue_in(0, 0, slab0, 0)

    def body(i, _):
        for js in range(2):
            S = slab[js]
            for j in range(SLAB):
                c = 10 * i + 5 * js + j
                b = (js + j) % 2
                wait_in(c, b, S, j)
                issue_out(b, S, j)
                if js == 0 and j == 0:
                    @pl.when(i > 0)
                    def _():
                        wait_out(1 - b)
                else:
                    wait_out(1 - b)
                if js == 0 and j == 1:
                    issue_slab(2 * i + 1, 1)
                if js == 1 and j == 1:
                    @pl.when(i < NBODY - 1)
                    def _():
                        issue_slab(2 * i + 2, 0)
                if j == SLAB - 1:
                    if js == 0:
                        wait_slab(2 * i + 1, 1)
                        issue_in(c + 1, 1 - b, slab1, 0)
                    else:
                        @pl.when(i < NBODY - 1)
                        def _():
                            wait_slab(2 * i + 2, 0)
                            issue_in(c + 1, 1 - b, slab0, 0)
                else:
                    issue_in(c + 1, 1 - b, S, j + 1)
        return 0

    lax.fori_loop(0, NBODY, body, 0)

    wait_out(1)

    plsc.subcore_barrier()

    pltpu.sync_copy(acc_x.at[pl.ds(row0, FLUSH)],
                    accx_out.at[cid, pl.ds(row0, FLUSH)])
    pltpu.sync_copy(acc_a.at[pl.ds(row0, FLUSH)],
                    acca_out.at[cid, pl.ds(row0, FLUSH)])

    @pl.when(sid == NS - 1)
    def _flush_tail():
        pltpu.sync_copy(acc_x.at[pl.ds(TAIL0, TAILR)],
                        accx_out.at[cid, pl.ds(TAIL0, TAILR)])
        pltpu.sync_copy(acc_a.at[pl.ds(TAIL0, TAILR)],
                        acca_out.at[cid, pl.ds(TAIL0, TAILR)])


_sc_segsum = pl.kernel(
    _sc_body,
    out_type=(
        jax.ShapeDtypeStruct((NC, N, D_FEAT), jnp.float32),
        jax.ShapeDtypeStruct((NC, N, D_AUG), jnp.float32),
    ),
    mesh=plsc.VectorSubcoreMesh(core_axis_name="c", subcore_axis_name="s"),
    scratch_types=[
        pltpu.VMEM((2, SPAN), jnp.int32),
        pltpu.VMEM((2, SPAN), jnp.int32),
        pltpu.VMEM((CHUNK, D_FEAT), jnp.float32),
        pltpu.VMEM((CHUNK, D_FEAT), jnp.float32),
        pltpu.VMEM((CHUNK, D_AUG), jnp.float32),
        pltpu.VMEM((CHUNK, D_AUG), jnp.float32),
        pltpu.VMEM_SHARED((N, D_FEAT), jnp.float32),
        pltpu.VMEM_SHARED((N, D_AUG), jnp.float32),
        pltpu.SemaphoreType.DMA,
        pltpu.SemaphoreType.DMA,
        pltpu.SemaphoreType.DMA,
        pltpu.SemaphoreType.DMA,
        pltpu.SemaphoreType.DMA,
        pltpu.SemaphoreType.DMA,
    ],
    compiler_params=pltpu.CompilerParams(use_tc_tiling_on_sc=False),
)


def _tc_body(x_ref, ax_ref, aa_ref, w1_ref, w2_ref, w5_ref, b1_ref, o_ref):
    acc = jnp.dot(x_ref[...], w1_ref[...], preferred_element_type=jnp.float32)
    acc += jnp.dot(ax_ref[0] + ax_ref[1], w2_ref[...],
                   preferred_element_type=jnp.float32)
    acc += jnp.dot(aa_ref[0] + aa_ref[1], w5_ref[...],
                   preferred_element_type=jnp.float32)
    o_ref[...] = acc + b1_ref[...]


ROW_BLK = 1000


def _tc_combine(x, accx, acca, W1, W2, W5aug, b1):
    return pl.pallas_call(
        _tc_body,
        out_shape=jax.ShapeDtypeStruct((N, D_FEAT), jnp.float32),
        grid=(N // ROW_BLK,),
        in_specs=[
            pl.BlockSpec((ROW_BLK, D_FEAT), lambda i: (i, 0)),
            pl.BlockSpec((NC, ROW_BLK, D_FEAT), lambda i: (0, i, 0)),
            pl.BlockSpec((NC, ROW_BLK, D_AUG), lambda i: (0, i, 0)),
            pl.BlockSpec((D_FEAT, D_FEAT), lambda i: (0, 0)),
            pl.BlockSpec((D_FEAT, D_FEAT), lambda i: (0, 0)),
            pl.BlockSpec((D_AUG, D_FEAT), lambda i: (0, 0)),
            pl.BlockSpec((1, D_FEAT), lambda i: (0, 0)),
        ],
        out_specs=pl.BlockSpec((ROW_BLK, D_FEAT), lambda i: (i, 0)),
    )(x, accx, acca, W1, W2, W5aug, b1)


def kernel(x, edge_index, edge_attr, W1, b1, W2, b2, W3, b3, W4, b4, W5, b5):
    accx, acca = _sc_segsum(x, edge_index, edge_attr)
    W5aug = jnp.zeros((D_AUG, D_FEAT), jnp.float32).at[0:16].set(W5).at[16].set(b2 + b5)
    return _tc_combine(x, accx, acca, W1, W2, W5aug, b1.reshape(1, D_FEAT))
